# Initial kernel scaffold; baseline (speedup 1.0000x reference)
import jax
import jax.numpy as jnp
from jax.experimental import pallas as pl


def kernel(X, table):
    raise NotImplementedError("write your pallas kernel here")


# speedup vs baseline: 1.1026x; 1.1026x over previous
import functools
import jax
import jax.numpy as jnp
from jax import lax
from jax.experimental import pallas as pl
from jax.experimental.pallas import tpu as pltpu
from jax.experimental.pallas import tpu_sc as plsc


@functools.lru_cache(maxsize=None)
def _make_gather(N, D, NC, NS, chunk):
    NW = NC * NS
    n_per_w = N // NW
    n_chunks = n_per_w // chunk
    mesh = plsc.VectorSubcoreMesh(core_axis_name="c", subcore_axis_name="s")

    @functools.partial(
        pl.kernel,
        mesh=mesh,
        out_type=jax.ShapeDtypeStruct((N, D), jnp.float32),
        scratch_types=[
            pltpu.VMEM((chunk,), jnp.int32),
            pltpu.VMEM((chunk, D), jnp.float32),
            pltpu.SemaphoreType.DMA,
        ],
        compiler_params=pltpu.CompilerParams(use_tc_tiling_on_sc=False),
    )
    def k(idx_hbm, table_hbm, out_hbm, idx_v, rows_v, sem):
        wid = lax.axis_index("s") * NC + lax.axis_index("c")
        base = wid * n_per_w

        def body(i, carry):
            off = base + i * chunk
            pltpu.sync_copy(idx_hbm.at[pl.ds(off, chunk)], idx_v)
            pltpu.async_copy(table_hbm.at[idx_v], rows_v, sem).wait()
            pltpu.sync_copy(rows_v, out_hbm.at[pl.ds(off, chunk)])
            return carry

        lax.fori_loop(0, n_chunks, body, 0)

    return k


def kernel(X, table):
    B, H = X.shape
    V, D = table.shape
    N = B * H
    info = plsc.get_sparse_core_info()
    flat = X.reshape(N).astype(jnp.int32)
    out = _make_gather(N, D, info.num_cores, info.num_subcores, 1600)(flat, table)
    return out.reshape(B, H, D)


# speedup vs baseline: 1.1098x; 1.0065x over previous
import functools
import jax
import jax.numpy as jnp
from jax import lax
from jax.experimental import pallas as pl
from jax.experimental.pallas import tpu as pltpu
from jax.experimental.pallas import tpu_sc as plsc


@functools.lru_cache(maxsize=None)
def _make_gather(N, D, NC, NS, chunk):
    NW = NC * NS
    n_per_w = N // NW
    n_chunks = n_per_w // chunk
    mesh = plsc.VectorSubcoreMesh(core_axis_name="c", subcore_axis_name="s")

    @functools.partial(
        pl.kernel,
        mesh=mesh,
        out_type=jax.ShapeDtypeStruct((N, D), jnp.float32),
        scratch_types=[
            pltpu.VMEM((n_per_w,), jnp.int32),
            pltpu.VMEM((2, chunk, D), jnp.float32),
            pltpu.SemaphoreType.DMA,
            pltpu.SemaphoreType.DMA,
            pltpu.SemaphoreType.DMA,
            pltpu.SemaphoreType.DMA,
        ],
        compiler_params=pltpu.CompilerParams(use_tc_tiling_on_sc=False),
    )
    def k(idx_hbm, table_hbm, out_hbm, idx_all, rows_v, sg0, sg1, ss0, ss1):
        wid = lax.axis_index("s") * NC + lax.axis_index("c")
        base = wid * n_per_w
        sem_g = [sg0, sg1]
        sem_s = [ss0, ss1]

        pltpu.sync_copy(idx_hbm.at[pl.ds(base, n_per_w)], idx_all)

        def gather_desc(c, b):
            return pltpu.make_async_copy(
                table_hbm.at[idx_all.at[pl.ds(c * chunk, chunk)]],
                rows_v.at[b], sem_g[b])

        def store_desc(c, b):
            return pltpu.make_async_copy(
                rows_v.at[b], out_hbm.at[pl.ds(base + c * chunk, chunk)],
                sem_s[b])

        gather_desc(0, 0).start()
        for c in range(n_chunks):
            b = c % 2
            b1 = (c + 1) % 2
            gather_desc(c, b).wait()
            if c + 1 < n_chunks:
                if c >= 1:
                    store_desc(c - 1, b1).wait()
                gather_desc(c + 1, b1).start()
            store_desc(c, b).start()
        store_desc(n_chunks - 1, (n_chunks - 1) % 2).wait()
        if n_chunks >= 2:
            store_desc(n_chunks - 2, (n_chunks - 2) % 2).wait()

    return k


def kernel(X, table):
    B, H = X.shape
    V, D = table.shape
    N = B * H
    info = plsc.get_sparse_core_info()
    flat = X.reshape(N).astype(jnp.int32)
    out = _make_gather(N, D, info.num_cores, info.num_subcores, 1600)(flat, table)
    return out.reshape(B, H, D)


# speedup vs baseline: 1.7808x; 1.6046x over previous
import functools
import jax
import jax.numpy as jnp
from jax import lax
from jax.experimental import pallas as pl
from jax.experimental.pallas import tpu as pltpu
from jax.experimental.pallas import tpu_sc as plsc


@functools.lru_cache(maxsize=None)
def _make_gather(B, H, D, NC, NS, chunk_b):
    NW = NC * NS
    b_per_w = B // NW
    n_per_w = b_per_w * H
    chunk = chunk_b * H
    n_chunks = b_per_w // chunk_b
    mesh = plsc.VectorSubcoreMesh(core_axis_name="c", subcore_axis_name="s")

    @functools.partial(
        pl.kernel,
        mesh=mesh,
        out_type=jax.ShapeDtypeStruct((B, H, D), jnp.float32),
        scratch_types=[
            pltpu.VMEM((b_per_w, H), jnp.int32),
            pltpu.VMEM((n_per_w,), jnp.int32),
            pltpu.VMEM((2, chunk, D), jnp.float32),
            pltpu.SemaphoreType.DMA,
            pltpu.SemaphoreType.DMA,
            pltpu.SemaphoreType.DMA,
            pltpu.SemaphoreType.DMA,
        ],
        compiler_params=pltpu.CompilerParams(
            use_tc_tiling_on_sc=False, needs_layout_passes=False),
    )
    def k(x_hbm, table_hbm, out_hbm, xv, idx_all, rows_v, sg0, sg1, ss0, ss1):
        wid = lax.axis_index("s") * NC + lax.axis_index("c")
        b_base = wid * b_per_w
        sem_g = [sg0, sg1]
        sem_s = [ss0, ss1]

        pltpu.sync_copy(x_hbm.at[pl.ds(b_base, b_per_w)], xv)

        def compact(g, carry):
            v = lax.iota(jnp.int32, 16) + g * 16
            t = plsc.load_gather(xv, [v // H, v % H])
            idx_all[pl.ds(g * 16, 16)] = t
            return carry

        lax.fori_loop(0, n_per_w // 16, compact, 0)

        def gather_desc(c, b):
            return pltpu.make_async_copy(
                table_hbm.at[idx_all.at[pl.ds(c * chunk, chunk)]],
                rows_v.at[b], sem_g[b])

        def start_stores(c, b):
            def one(j, carry):
                pltpu.make_async_copy(
                    rows_v.at[b].at[pl.ds(j * H, H)],
                    out_hbm.at[b_base + c * chunk_b + j], sem_s[b]).start()
                return carry
            lax.fori_loop(0, chunk_b, one, 0)

        def drain_stores(b):
            pltpu.make_async_copy(
                table_hbm.at[pl.ds(0, chunk)], rows_v.at[b], sem_s[b]).wait()

        gather_desc(0, 0).start()
        for c in range(n_chunks):
            b = c % 2
            b1 = (c + 1) % 2
            gather_desc(c, b).wait()
            if c + 1 < n_chunks:
                if c >= 1:
                    drain_stores(b1)
                gather_desc(c + 1, b1).start()
            start_stores(c, b)
        drain_stores(n_chunks % 2)
        drain_stores((n_chunks - 1) % 2)

    return k


def kernel(X, table):
    B, H = X.shape
    V, D = table.shape
    info = plsc.get_sparse_core_info()
    return _make_gather(B, H, D, info.num_cores, info.num_subcores, 16)(X, table)
